# baseline (speedup 1.0000x reference)
import functools

import jax
import jax.numpy as jnp
from jax import lax
from jax.experimental import pallas as pl
from jax.experimental.pallas import tpu as pltpu
from jax.experimental.pallas import tpu_sc as plsc

F = 26
CARD = 100000
E = 32
H = 32
O = 16
B = 4096
FB = F * B
ROWS = F * E

NC = 2
NS = 16
NW = NC * NS
NCHUNKS = 1
FC = F // NCHUNKS
CROWS = FC * E
RPW = CROWS // NW
LANES = 16
NVEC_B = B // LANES
NVEC_H = 2 * B // LANES


HALF = CARD // 2


def _sc_gather_body(row_base, idx_hbm, table_hbm, out_hbm, idx2_v, seg_a, seg_b, out2_v, sem_a, sem_b, sem_out):
    wid = lax.axis_index("s") * NC + lax.axis_index("c")
    r0 = row_base + wid * RPW
    f0 = r0 // E
    f1 = jnp.minimum(f0 + 1, F - 1)

    pltpu.sync_copy(idx_hbm.at[pl.ds(f0 * B, B)], idx2_v.at[pl.ds(0, B)])
    pltpu.sync_copy(idx_hbm.at[pl.ds(f1 * B, B)], idx2_v.at[pl.ds(B, B)])

    @plsc.parallel_loop(0, NVEC_H, unroll=4)
    def hash_body(i):
        off = i * LANES
        x = idx2_v[pl.ds(off, LANES)]
        idx2_v[pl.ds(off, LANES)] = lax.rem(x + 1, CARD)

    segs = [seg_a, seg_b]
    sems = [sem_a, sem_b]
    NSEG = 2 * RPW

    def seg_fetch(s):
        return pltpu.async_copy(
            table_hbm.at[pl.ds((r0 + s // 2) * CARD + (s % 2) * HALF, HALF)],
            segs[s % 2],
            sems[s % 2],
        )

    in_descs = [seg_fetch(0), seg_fetch(1)]
    out_descs = [None, None]
    for s in range(NSEG):
        k = s // 2
        r = r0 + k
        f = r // E
        selbase = (f - f0) * B
        kk = k % 2
        buf = segs[s % 2]
        in_descs[s % 2].wait()
        if s % 2 == 0:
            if out_descs[kk] is not None:
                out_descs[kk].wait()

            @plsc.parallel_loop(0, NVEC_B, unroll=8)
            def gather_lo(i):
                off = i * LANES
                idxs = idx2_v[pl.ds(selbase + off, LANES)]
                out2_v[pl.ds(kk * B + off, LANES)] = plsc.load_gather(
                    buf, [jnp.minimum(idxs, HALF - 1)]
                )
        else:
            @plsc.parallel_loop(0, NVEC_B, unroll=8)
            def gather_hi(i):
                off = i * LANES
                idxs = idx2_v[pl.ds(selbase + off, LANES)]
                g = plsc.load_gather(buf, [jnp.maximum(idxs - HALF, 0)])
                prev = out2_v[pl.ds(kk * B + off, LANES)]
                out2_v[pl.ds(kk * B + off, LANES)] = jnp.where(
                    idxs >= HALF, g, prev
                )
            out_descs[kk] = pltpu.async_copy(
                out2_v.at[pl.ds(kk * B, B)], out_hbm.at[r - row_base], sem_out
            )
        if s + 2 < NSEG:
            in_descs[s % 2] = seg_fetch(s + 2)
    for d in out_descs:
        if d is not None:
            d.wait()


@functools.cache
def _make_sc_gather(row_base):
    mesh = plsc.VectorSubcoreMesh(core_axis_name="c", subcore_axis_name="s")
    return functools.partial(
        pl.kernel,
        mesh=mesh,
        compiler_params=pltpu.CompilerParams(needs_layout_passes=False),
        out_type=jax.ShapeDtypeStruct((CROWS, B), jnp.float32),
        scratch_types=[
            pltpu.VMEM((2 * B,), jnp.int32),
            pltpu.VMEM((HALF,), jnp.float32),
            pltpu.VMEM((HALF,), jnp.float32),
            pltpu.VMEM((2 * B,), jnp.float32),
            pltpu.SemaphoreType.DMA,
            pltpu.SemaphoreType.DMA,
            pltpu.SemaphoreType.DMA,
        ],
    )(functools.partial(_sc_gather_body, row_base))


BT = 4096
NB = B // BT


def _mlp_body(embT_ref, w1_ref, b1_ref, w2_ref, b2_ref, out_ref):
    eT = embT_ref[0]
    h = lax.dot_general(w1_ref[0], eT, (((0,), (0,)), ((), ())),
                        preferred_element_type=jnp.float32)
    h = jnp.maximum(h + b1_ref[0], 0.0)
    out_ref[0] = (
        lax.dot_general(w2_ref[0], h, (((0,), (0,)), ((), ())),
                        preferred_element_type=jnp.float32)
        + b2_ref[0]
    )


_mlp = pl.pallas_call(
    _mlp_body,
    grid=(FC, NB),
    in_specs=[
        pl.BlockSpec((1, E, BT), lambda f, b: (f, 0, b)),
        pl.BlockSpec((1, E, H), lambda f, b: (f, 0, 0)),
        pl.BlockSpec((1, H, 1), lambda f, b: (f, 0, 0)),
        pl.BlockSpec((1, H, O), lambda f, b: (f, 0, 0)),
        pl.BlockSpec((1, O, 1), lambda f, b: (f, 0, 0)),
    ],
    out_specs=pl.BlockSpec((1, O, BT), lambda f, b: (f, 0, b)),
    out_shape=jax.ShapeDtypeStruct((FC, O, B), jnp.float32),
)


def kernel(inputs, tables, W1, b1, W2, b2):
    idx_flat = inputs.T.reshape(FB)
    table_rows = jnp.swapaxes(tables, 1, 2).reshape(ROWS * CARD)
    outs = []
    for c in range(NCHUNKS):
        embT = _make_sc_gather(c * CROWS)(idx_flat, table_rows)
        embT3 = embT.reshape(FC, E, B)
        fsl = slice(c * FC, (c + 1) * FC)
        outs.append(
            _mlp(embT3, W1[fsl], b1[fsl].reshape(FC, H, 1),
                 W2[fsl], b2[fsl].reshape(FC, O, 1))
        )
    outT = jnp.concatenate(outs, axis=0)
    return jnp.swapaxes(outT, 1, 2)


# speedup vs baseline: 1.5664x; 1.5664x over previous
import functools

import jax
import jax.numpy as jnp
from jax import lax
from jax.experimental import pallas as pl
from jax.experimental.pallas import tpu as pltpu
from jax.experimental.pallas import tpu_sc as plsc

F = 26
CARD = 100000
E = 32
H = 32
O = 16
B = 4096
FB = F * B
ROWS = F * E

NC = 2
NS = 16
NW = NC * NS
NCHUNKS = 1
FC = F // NCHUNKS
CROWS = FC * E
RPW = CROWS // NW
LANES = 16
NVEC_B = B // LANES
NVEC_H = 2 * B // LANES


HALF = CARD // 2


def _sc_gather_body(row_base, idx_hbm, table_hbm, out_hbm, idx2_v, seg_a, seg_b, out2_v, sem_a, sem_b, sem_out):
    wid = lax.axis_index("s") * NC + lax.axis_index("c")
    r0 = row_base + wid * RPW
    f0 = r0 // E
    f1 = jnp.minimum(f0 + 1, F - 1)

    pltpu.sync_copy(idx_hbm.at[pl.ds(f0 * B, B)], idx2_v.at[pl.ds(0, B)])
    pltpu.sync_copy(idx_hbm.at[pl.ds(f1 * B, B)], idx2_v.at[pl.ds(B, B)])

    @plsc.parallel_loop(0, NVEC_H, unroll=4)
    def hash_body(i):
        off = i * LANES
        x = idx2_v[pl.ds(off, LANES)]
        idx2_v[pl.ds(off, LANES)] = lax.rem(x + 1, CARD)

    segs = [seg_a, seg_b]
    sems = [sem_a, sem_b]
    NSEG = 2 * RPW

    def seg_fetch(s):
        return pltpu.async_copy(
            table_hbm.at[(r0 + s // 2) * 2 + s % 2],
            segs[s % 2],
            sems[s % 2],
        )

    in_descs = [seg_fetch(0), seg_fetch(1)]
    out_descs = [None, None]
    for s in range(NSEG):
        k = s // 2
        r = r0 + k
        f = r // E
        selbase = (f - f0) * B
        kk = k % 2
        buf = segs[s % 2]
        in_descs[s % 2].wait()
        if s % 2 == 0:
            if out_descs[kk] is not None:
                out_descs[kk].wait()

            @plsc.parallel_loop(0, NVEC_B, unroll=8)
            def gather_lo(i):
                off = i * LANES
                idxs = idx2_v[pl.ds(selbase + off, LANES)]
                out2_v[pl.ds(kk * B + off, LANES)] = plsc.load_gather(
                    buf, [jnp.minimum(idxs, HALF - 1)]
                )
        else:
            @plsc.parallel_loop(0, NVEC_B, unroll=8)
            def gather_hi(i):
                off = i * LANES
                idxs = idx2_v[pl.ds(selbase + off, LANES)]
                g = plsc.load_gather(buf, [jnp.maximum(idxs - HALF, 0)])
                prev = out2_v[pl.ds(kk * B + off, LANES)]
                out2_v[pl.ds(kk * B + off, LANES)] = jnp.where(
                    idxs >= HALF, g, prev
                )
            out_descs[kk] = pltpu.async_copy(
                out2_v.at[pl.ds(kk * B, B)], out_hbm.at[r - row_base], sem_out
            )
        if s + 2 < NSEG:
            in_descs[s % 2] = seg_fetch(s + 2)
    for d in out_descs:
        if d is not None:
            d.wait()


@functools.cache
def _make_sc_gather(row_base):
    mesh = plsc.VectorSubcoreMesh(core_axis_name="c", subcore_axis_name="s")
    return functools.partial(
        pl.kernel,
        mesh=mesh,
        compiler_params=pltpu.CompilerParams(needs_layout_passes=False),
        out_type=jax.ShapeDtypeStruct((CROWS, B), jnp.float32),
        scratch_types=[
            pltpu.VMEM((2 * B,), jnp.int32),
            pltpu.VMEM((HALF,), jnp.float32),
            pltpu.VMEM((HALF,), jnp.float32),
            pltpu.VMEM((2 * B,), jnp.float32),
            pltpu.SemaphoreType.DMA,
            pltpu.SemaphoreType.DMA,
            pltpu.SemaphoreType.DMA,
        ],
    )(functools.partial(_sc_gather_body, row_base))


BT = 4096
NB = B // BT


def _mlp_body(embT_ref, w1_ref, b1_ref, w2_ref, b2_ref, out_ref):
    eT = embT_ref[0]
    h = lax.dot_general(w1_ref[0], eT, (((0,), (0,)), ((), ())),
                        preferred_element_type=jnp.float32)
    h = jnp.maximum(h + b1_ref[0], 0.0)
    out_ref[0] = (
        lax.dot_general(w2_ref[0], h, (((0,), (0,)), ((), ())),
                        preferred_element_type=jnp.float32)
        + b2_ref[0]
    )


_mlp = pl.pallas_call(
    _mlp_body,
    grid=(FC, NB),
    in_specs=[
        pl.BlockSpec((1, E, BT), lambda f, b: (f, 0, b)),
        pl.BlockSpec((1, E, H), lambda f, b: (f, 0, 0)),
        pl.BlockSpec((1, H, 1), lambda f, b: (f, 0, 0)),
        pl.BlockSpec((1, H, O), lambda f, b: (f, 0, 0)),
        pl.BlockSpec((1, O, 1), lambda f, b: (f, 0, 0)),
    ],
    out_specs=pl.BlockSpec((1, O, BT), lambda f, b: (f, 0, b)),
    out_shape=jax.ShapeDtypeStruct((FC, O, B), jnp.float32),
)


def kernel(inputs, tables, W1, b1, W2, b2):
    idx_flat = inputs.T.reshape(FB)
    table_rows = jnp.swapaxes(tables, 1, 2).reshape(ROWS * 2, HALF)
    outs = []
    for c in range(NCHUNKS):
        embT = _make_sc_gather(c * CROWS)(idx_flat, table_rows)
        embT3 = embT.reshape(FC, E, B)
        fsl = slice(c * FC, (c + 1) * FC)
        outs.append(
            _mlp(embT3, W1[fsl], b1[fsl].reshape(FC, H, 1),
                 W2[fsl], b2[fsl].reshape(FC, O, 1))
        )
    outT = jnp.concatenate(outs, axis=0)
    return jnp.swapaxes(outT, 1, 2)


# speedup vs baseline: 2.9403x; 1.8771x over previous
import functools

import jax
import jax.numpy as jnp
from jax import lax
from jax.experimental import pallas as pl
from jax.experimental.pallas import tpu as pltpu
from jax.experimental.pallas import tpu_sc as plsc

F = 26
CARD = 100000
E = 32
H = 32
O = 16
B = 4096
FB = F * B
ROWS = F * E

NC = 2
NS = 16
NW = NC * NS
NCHUNKS = 2
FC = F // NCHUNKS
CROWS = FC * E
RPW = CROWS // NW
LANES = 16
NVEC_B = B // LANES
NVEC_H = 2 * B // LANES


def _sc_gather_body(row_base, idx_hbm, table_hbm, out_hbm, idx2_v, row_v, out2_v, sem, sem_out):
    wid = lax.axis_index("s") * NC + lax.axis_index("c")
    r0 = row_base + wid * RPW
    f0 = r0 // E
    f1 = jnp.minimum(f0 + 1, F - 1)

    pltpu.sync_copy(idx_hbm.at[pl.ds(f0 * B, B)], idx2_v.at[pl.ds(0, B)])
    pltpu.sync_copy(idx_hbm.at[pl.ds(f1 * B, B)], idx2_v.at[pl.ds(B, B)])

    @plsc.parallel_loop(0, NVEC_H, unroll=4)
    def hash_body(i):
        off = i * LANES
        x = idx2_v[pl.ds(off, LANES)]
        idx2_v[pl.ds(off, LANES)] = lax.rem(x + 1, CARD)

    out_descs = [None, None]
    for k in range(RPW):
        r = r0 + k
        f = r // E
        selbase = (f - f0) * B
        pltpu.sync_copy(table_hbm.at[r], row_v)
        kk = k % 2
        if out_descs[kk] is not None:
            out_descs[kk].wait()

        @plsc.parallel_loop(0, NVEC_B, unroll=8)
        def gather_body(i):
            off = i * LANES
            idxs = idx2_v[pl.ds(selbase + off, LANES)]
            out2_v[pl.ds(kk * B + off, LANES)] = plsc.load_gather(row_v, [idxs])

        out_descs[kk] = pltpu.async_copy(
            out2_v.at[pl.ds(kk * B, B)], out_hbm.at[r - row_base], sem_out
        )
    for d in out_descs:
        if d is not None:
            d.wait()


@functools.cache
def _make_sc_gather(row_base):
    mesh = plsc.VectorSubcoreMesh(core_axis_name="c", subcore_axis_name="s")
    return functools.partial(
        pl.kernel,
        mesh=mesh,
        compiler_params=pltpu.CompilerParams(needs_layout_passes=False),
        out_type=jax.ShapeDtypeStruct((CROWS, B), jnp.float32),
        scratch_types=[
            pltpu.VMEM((2 * B,), jnp.int32),
            pltpu.VMEM((CARD,), jnp.float32),
            pltpu.VMEM((2 * B,), jnp.float32),
            pltpu.SemaphoreType.DMA,
            pltpu.SemaphoreType.DMA,
        ],
    )(functools.partial(_sc_gather_body, row_base))


BT = 4096
NB = B // BT


def _mlp_body(embT_ref, w1_ref, b1_ref, w2_ref, b2_ref, out_ref):
    eT = embT_ref[0]
    h = lax.dot_general(w1_ref[0], eT, (((0,), (0,)), ((), ())),
                        preferred_element_type=jnp.float32)
    h = jnp.maximum(h + b1_ref[0], 0.0)
    out_ref[0] = (
        lax.dot_general(w2_ref[0], h, (((0,), (0,)), ((), ())),
                        preferred_element_type=jnp.float32)
        + b2_ref[0]
    )


_mlp = pl.pallas_call(
    _mlp_body,
    grid=(FC, NB),
    in_specs=[
        pl.BlockSpec((1, E, BT), lambda f, b: (f, 0, b)),
        pl.BlockSpec((1, E, H), lambda f, b: (f, 0, 0)),
        pl.BlockSpec((1, H, 1), lambda f, b: (f, 0, 0)),
        pl.BlockSpec((1, H, O), lambda f, b: (f, 0, 0)),
        pl.BlockSpec((1, O, 1), lambda f, b: (f, 0, 0)),
    ],
    out_specs=pl.BlockSpec((1, O, BT), lambda f, b: (f, 0, b)),
    out_shape=jax.ShapeDtypeStruct((FC, O, B), jnp.float32),
)


def kernel(inputs, tables, W1, b1, W2, b2):
    idx_flat = inputs.T.reshape(FB)
    table_rows = jnp.swapaxes(tables, 1, 2).reshape(ROWS, CARD)
    outs = []
    for c in range(NCHUNKS):
        embT = _make_sc_gather(c * CROWS)(idx_flat, table_rows)
        embT3 = embT.reshape(FC, E, B)
        fsl = slice(c * FC, (c + 1) * FC)
        outs.append(
            _mlp(embT3, W1[fsl], b1[fsl].reshape(FC, H, 1),
                 W2[fsl], b2[fsl].reshape(FC, O, 1))
        )
    outT = jnp.concatenate(outs, axis=0)
    return jnp.swapaxes(outT, 1, 2)


# speedup vs baseline: 3.2903x; 1.1190x over previous
import functools

import jax
import jax.numpy as jnp
from jax import lax
from jax.experimental import pallas as pl
from jax.experimental.pallas import tpu as pltpu
from jax.experimental.pallas import tpu_sc as plsc

F = 26
CARD = 100000
E = 32
H = 32
O = 16
B = 4096
FB = F * B
ROWS = F * E

NC = 2
NS = 16
NW = NC * NS
NCHUNKS = 1
FC = F // NCHUNKS
CROWS = FC * E
RPW = CROWS // NW
LANES = 16
NVEC_B = B // LANES
NVEC_H = 2 * B // LANES


def _sc_gather_body(row_base, idx_hbm, table_hbm, out_hbm, idx2_v, row_v, out2_v, sem, sem_out):
    wid = lax.axis_index("s") * NC + lax.axis_index("c")
    r0 = row_base + wid * RPW
    f0 = r0 // E
    f1 = jnp.minimum(f0 + 1, F - 1)

    pltpu.sync_copy(idx_hbm.at[pl.ds(f0 * B, B)], idx2_v.at[pl.ds(0, B)])
    pltpu.sync_copy(idx_hbm.at[pl.ds(f1 * B, B)], idx2_v.at[pl.ds(B, B)])

    @plsc.parallel_loop(0, NVEC_H, unroll=4)
    def hash_body(i):
        off = i * LANES
        x = idx2_v[pl.ds(off, LANES)]
        idx2_v[pl.ds(off, LANES)] = lax.rem(x + 1, CARD)

    out_descs = [None, None]
    for k in range(RPW):
        r = r0 + k
        f = r // E
        selbase = (f - f0) * B
        pltpu.sync_copy(table_hbm.at[r], row_v)
        kk = k % 2
        if out_descs[kk] is not None:
            out_descs[kk].wait()

        @plsc.parallel_loop(0, NVEC_B, unroll=8)
        def gather_body(i):
            off = i * LANES
            idxs = idx2_v[pl.ds(selbase + off, LANES)]
            out2_v[pl.ds(kk * B + off, LANES)] = plsc.load_gather(row_v, [idxs])

        out_descs[kk] = pltpu.async_copy(
            out2_v.at[pl.ds(kk * B, B)], out_hbm.at[r - row_base], sem_out
        )
    for d in out_descs:
        if d is not None:
            d.wait()


@functools.cache
def _make_sc_gather(row_base):
    mesh = plsc.VectorSubcoreMesh(core_axis_name="c", subcore_axis_name="s")
    return functools.partial(
        pl.kernel,
        mesh=mesh,
        compiler_params=pltpu.CompilerParams(needs_layout_passes=False),
        out_type=jax.ShapeDtypeStruct((CROWS, B), jnp.float32),
        scratch_types=[
            pltpu.VMEM((2 * B,), jnp.int32),
            pltpu.VMEM((CARD,), jnp.float32),
            pltpu.VMEM((2 * B,), jnp.float32),
            pltpu.SemaphoreType.DMA,
            pltpu.SemaphoreType.DMA,
        ],
    )(functools.partial(_sc_gather_body, row_base))


BT = 4096
NB = B // BT


def _mlp_body(embT_ref, w1_ref, b1_ref, w2_ref, b2_ref, out_ref):
    eT = embT_ref[0]
    h = lax.dot_general(w1_ref[0], eT, (((0,), (0,)), ((), ())),
                        preferred_element_type=jnp.float32)
    h = jnp.maximum(h + b1_ref[0], 0.0)
    out_ref[0] = (
        lax.dot_general(w2_ref[0], h, (((0,), (0,)), ((), ())),
                        preferred_element_type=jnp.float32)
        + b2_ref[0]
    )


_mlp = pl.pallas_call(
    _mlp_body,
    grid=(FC, NB),
    in_specs=[
        pl.BlockSpec((1, E, BT), lambda f, b: (f, 0, b)),
        pl.BlockSpec((1, E, H), lambda f, b: (f, 0, 0)),
        pl.BlockSpec((1, H, 1), lambda f, b: (f, 0, 0)),
        pl.BlockSpec((1, H, O), lambda f, b: (f, 0, 0)),
        pl.BlockSpec((1, O, 1), lambda f, b: (f, 0, 0)),
    ],
    out_specs=pl.BlockSpec((1, O, BT), lambda f, b: (f, 0, b)),
    out_shape=jax.ShapeDtypeStruct((FC, O, B), jnp.float32),
)


def kernel(inputs, tables, W1, b1, W2, b2):
    idx_flat = inputs.T.reshape(FB)
    table_rows = jnp.swapaxes(tables, 1, 2).reshape(ROWS, CARD)
    outs = []
    for c in range(NCHUNKS):
        embT = _make_sc_gather(c * CROWS)(idx_flat, table_rows)
        embT3 = embT.reshape(FC, E, B)
        fsl = slice(c * FC, (c + 1) * FC)
        outs.append(
            _mlp(embT3, W1[fsl], b1[fsl].reshape(FC, H, 1),
                 W2[fsl], b2[fsl].reshape(FC, O, 1))
        )
    outT = jnp.concatenate(outs, axis=0)
    return jnp.swapaxes(outT, 1, 2)
